# baseline (speedup 1.0000x reference)
import functools

import jax
import jax.numpy as jnp
from jax.experimental import pallas as pl
from jax.experimental.pallas import tpu as pltpu

LAMBDA_REG = 1.0
LAMBDA_CLS = 1.0
NEG_POS_RATIO = 3.0
IGNORE_LABEL = 2
POS_LABEL = 1
NEG_LABEL = 0
ALPHA, GAMMA, BETA = 0.5, 1.5, 1.0

_MIN32 = -2147483648


def _softplus(u):
    return jnp.maximum(u, 0.0) + jnp.log1p(jnp.exp(-jnp.abs(u)))


def _balanced_l1(diff):
    b = 2.718281828459045 ** (GAMMA / ALPHA) - 1.0
    return jnp.where(
        diff < BETA,
        ALPHA / b * (b * diff + 1.0) * jnp.log(b * diff / BETA + 1.0) - ALPHA * diff,
        GAMMA * diff + GAMMA / b - ALPHA * BETA,
    )


def _loss_kernel(pb_ref, gb_ref, pc_ref, mi_ref, loc_ref, conf_ref):
    f32 = jnp.float32

    mi = mi_ref[...]
    pos = mi == POS_LABEL
    neg = mi == NEG_LABEL

    diff = jnp.abs(pb_ref[...] - gb_ref[...])
    bl1 = _balanced_l1(diff)
    r512 = jax.lax.broadcasted_iota(jnp.int32, (512, 128), 0)
    c128 = jax.lax.broadcasted_iota(jnp.int32, (512, 128), 1)
    s4 = ((r512 >> 2) == c128).astype(f32)
    bl1_anchor = jax.lax.dot_general(
        bl1, s4, (((1,), (0,)), ((), ())), precision=jax.lax.Precision.HIGHEST
    )
    bl1_sum = jnp.sum(jnp.where(pos, bl1_anchor, 0.0))

    r256 = jax.lax.broadcasted_iota(jnp.int32, (256, 128), 0)
    c2 = jax.lax.broadcasted_iota(jnp.int32, (256, 128), 1)
    dmat = (r256 == 2 * c2 + 1).astype(f32) - (r256 == 2 * c2).astype(f32)
    d = jax.lax.dot_general(
        pc_ref[...], dmat, (((1,), (0,)), ((), ())), precision=jax.lax.Precision.HIGHEST
    )

    ce_neg = _softplus(d)
    ce_pos = _softplus(-d)

    pos_ce = jnp.where(pos, ce_pos, 0.0)
    pos_sum = jnp.sum(pos_ce)
    pos_nz = jnp.sum((pos & (ce_pos != 0.0)).astype(jnp.int32))

    nv = jnp.sum(pos.astype(jnp.int32))
    nneg = jnp.sum(neg.astype(jnp.int32))
    k = jnp.minimum(
        nv.astype(f32) * NEG_POS_RATIO, nneg.astype(f32)
    ).astype(jnp.int32)

    min32 = jnp.int32(_MIN32)
    bits = jax.lax.bitcast_convert_type(ce_neg, jnp.int32)
    s_keys = jnp.where(neg, bits, min32 + 1)

    def body(i, p_u):
        j = 31 - i
        c_u = p_u | jnp.left_shift(jnp.int32(1), j)
        c_s = c_u ^ min32
        cnt = jnp.sum((s_keys >= c_s).astype(jnp.int32))
        return jnp.where(cnt >= k, c_u, p_u)

    t_u = jax.lax.fori_loop(0, 32, body, jnp.int32(0))
    t_s = t_u ^ min32
    t_val = jax.lax.bitcast_convert_type(t_s, f32)

    gt_mask = s_keys > t_s
    cnt_gt = jnp.sum(gt_mask.astype(jnp.int32))
    sum_gt = jnp.sum(jnp.where(gt_mask, ce_neg, 0.0))

    kf = k.astype(f32)
    hard_sum = sum_gt + (kf - cnt_gt.astype(f32)) * t_val
    hard_nz = jnp.where(t_val > 0.0, k, cnt_gt)
    hard_sum = jnp.where(k > 0, hard_sum, 0.0)
    hard_nz = jnp.where(k > 0, hard_nz, 0)

    loc_ref[0, 0] = LAMBDA_REG * bl1_sum / nv.astype(f32)
    ns = (pos_nz + hard_nz).astype(f32)
    conf_ref[0, 0] = LAMBDA_CLS * (pos_sum + hard_sum) / ns


@jax.jit
def kernel(predicted_bboxes, predicted_classes, gt_bboxes, matching_indicators):
    B, A, C = predicted_classes.shape
    n = B * A
    rows = n // 128
    pb = predicted_bboxes.reshape(rows, 512)
    gb = gt_bboxes.reshape(rows, 512)
    pc = predicted_classes.reshape(rows, 256)
    mi = matching_indicators.reshape(rows, 128)

    loc, conf = pl.pallas_call(
        _loss_kernel,
        out_shape=[
            jax.ShapeDtypeStruct((1, 1), jnp.float32),
            jax.ShapeDtypeStruct((1, 1), jnp.float32),
        ],
        out_specs=[
            pl.BlockSpec(memory_space=pltpu.SMEM),
            pl.BlockSpec(memory_space=pltpu.SMEM),
        ],
    )(pb, gb, pc, mi)
    return (loc[0, 0], conf[0, 0])


# speedup vs baseline: 23.3936x; 23.3936x over previous
import jax
import jax.numpy as jnp
from jax.experimental import pallas as pl
from jax.experimental.pallas import tpu as pltpu

LAMBDA_REG = 1.0
LAMBDA_CLS = 1.0
NEG_POS_RATIO = 3.0
IGNORE_LABEL = 2
POS_LABEL = 1
NEG_LABEL = 0
ALPHA, GAMMA, BETA = 0.5, 1.5, 1.0

_MIN32 = -2147483648
_B = 16
_A = 32768
_CHUNK = 4096
_STEPS = _A // _CHUNK


def _softplus(u):
    return jnp.maximum(u, 0.0) + jnp.log1p(jnp.exp(-jnp.abs(u)))


def _balanced_l1(diff):
    b = 2.718281828459045 ** (GAMMA / ALPHA) - 1.0
    return jnp.where(
        diff < BETA,
        ALPHA / b * (b * diff + 1.0) * jnp.log(b * diff / BETA + 1.0) - ALPHA * diff,
        GAMMA * diff + GAMMA / b - ALPHA * BETA,
    )


def _loss_kernel(pb_ref, gb_ref, pc_ref, mi_ref, loc_ref, conf_ref,
                 keys_ref, facc_ref, iacc_ref):
    f32 = jnp.float32
    step = pl.program_id(0)
    min32 = jnp.int32(_MIN32)

    @pl.when(step == 0)
    def _init():
        facc_ref[0] = 0.0
        facc_ref[1] = 0.0
        iacc_ref[0] = 0
        iacc_ref[1] = 0
        iacc_ref[2] = 0

    @pl.when(step < _STEPS)
    def _accumulate():
        mi = mi_ref[...]
        pos = mi == POS_LABEL
        neg = mi == NEG_LABEL

        diff = jnp.abs(pb_ref[...] - gb_ref[...])
        bl1 = _balanced_l1(diff)
        bl1_anchor = bl1[0] + bl1[1] + bl1[2] + bl1[3]
        facc_ref[0] += jnp.sum(jnp.where(pos, bl1_anchor, 0.0))

        d = pc_ref[1] - pc_ref[0]
        ce_neg = _softplus(d)
        ce_pos = _softplus(-d)

        facc_ref[1] += jnp.sum(jnp.where(pos, ce_pos, 0.0))
        iacc_ref[0] += jnp.sum(pos.astype(jnp.int32))
        iacc_ref[1] += jnp.sum(neg.astype(jnp.int32))
        iacc_ref[2] += jnp.sum((pos & (ce_pos != 0.0)).astype(jnp.int32))

        bits = jax.lax.bitcast_convert_type(ce_neg, jnp.int32)
        col = step * _CHUNK
        keys_ref[:, pl.ds(col, _CHUNK)] = jnp.where(neg, bits, min32 + 1)

    @pl.when(step == _STEPS)
    def _select():
        nv = iacc_ref[0]
        nneg = iacc_ref[1]
        k = jnp.minimum(
            nv.astype(f32) * NEG_POS_RATIO, nneg.astype(f32)
        ).astype(jnp.int32)

        s_keys = keys_ref[...]

        def body(i, p_u):
            j = 31 - i
            c_u = p_u | jnp.left_shift(jnp.int32(1), j)
            c_s = c_u ^ min32
            cnt = jnp.sum((s_keys >= c_s).astype(jnp.int32))
            return jnp.where(cnt >= k, c_u, p_u)

        t_u = jax.lax.fori_loop(0, 32, body, jnp.int32(0))
        t_s = t_u ^ min32
        t_val = jax.lax.bitcast_convert_type(t_s, f32)

        gt_mask = s_keys > t_s
        cnt_gt = jnp.sum(gt_mask.astype(jnp.int32))
        vals = jax.lax.bitcast_convert_type(s_keys, f32)
        sum_gt = jnp.sum(jnp.where(gt_mask, vals, 0.0))

        kf = k.astype(f32)
        hard_sum = sum_gt + (kf - cnt_gt.astype(f32)) * t_val
        hard_nz = jnp.where(t_val > 0.0, k, cnt_gt)
        hard_sum = jnp.where(k > 0, hard_sum, 0.0)
        hard_nz = jnp.where(k > 0, hard_nz, 0)

        loc_ref[0, 0] = LAMBDA_REG * facc_ref[0] / nv.astype(f32)
        ns = (iacc_ref[2] + hard_nz).astype(f32)
        conf_ref[0, 0] = LAMBDA_CLS * (facc_ref[1] + hard_sum) / ns


@jax.jit
def kernel(predicted_bboxes, predicted_classes, gt_bboxes, matching_indicators):
    pbt = jnp.transpose(predicted_bboxes, (2, 0, 1))
    gbt = jnp.transpose(gt_bboxes, (2, 0, 1))
    pct = jnp.transpose(predicted_classes, (2, 0, 1))
    mi = matching_indicators

    def chunk(i):
        return jnp.minimum(i, _STEPS - 1)

    loc, conf = pl.pallas_call(
        _loss_kernel,
        grid=(_STEPS + 1,),
        in_specs=[
            pl.BlockSpec((4, _B, _CHUNK), lambda i: (0, 0, chunk(i))),
            pl.BlockSpec((4, _B, _CHUNK), lambda i: (0, 0, chunk(i))),
            pl.BlockSpec((2, _B, _CHUNK), lambda i: (0, 0, chunk(i))),
            pl.BlockSpec((_B, _CHUNK), lambda i: (0, chunk(i))),
        ],
        out_shape=[
            jax.ShapeDtypeStruct((1, 1), jnp.float32),
            jax.ShapeDtypeStruct((1, 1), jnp.float32),
        ],
        out_specs=[
            pl.BlockSpec(memory_space=pltpu.SMEM),
            pl.BlockSpec(memory_space=pltpu.SMEM),
        ],
        scratch_shapes=[
            pltpu.VMEM((_B, _A), jnp.int32),
            pltpu.SMEM((2,), jnp.float32),
            pltpu.SMEM((3,), jnp.int32),
        ],
    )(pbt, gbt, pct, mi)
    return (loc[0, 0], conf[0, 0])
